# baseline (speedup 1.0000x reference)
import functools

import jax
import jax.numpy as jnp
from jax import lax
from jax.experimental import pallas as pl
from jax.experimental.pallas import tpu as pltpu
from jax.experimental.pallas import tpu_sc as plsc

BATCH = 16384
EMB = 32

_info = plsc.get_sparse_core_info()
_NC = _info.num_cores
_NS = _info.num_subcores
_NW = _NC * _NS
_BPW = BATCH // _NW
_CHUNK = 128
_NCHUNK = _BPW // _CHUNK


@functools.partial(
    pl.kernel,
    mesh=plsc.VectorSubcoreMesh(core_axis_name="c", subcore_axis_name="s"),
    out_type=jax.ShapeDtypeStruct((BATCH, 2, EMB), jnp.float32),
    compiler_params=pltpu.CompilerParams(use_tc_tiling_on_sc=False),
    scratch_types=[
        pltpu.VMEM((_NCHUNK, _CHUNK), jnp.int32),
        pltpu.VMEM((_NCHUNK, _CHUNK), jnp.int32),
        pltpu.VMEM((_BPW, EMB), jnp.float32),
        pltpu.VMEM((_BPW, EMB), jnp.float32),
        pltpu.SemaphoreType.DMA,
        pltpu.SemaphoreType.DMA,
    ],
)
def _action_encoder(blk_idx_hbm, dir_idx_hbm, blk_tab_hbm, dir_tab_hbm,
                    out_hbm, bidx_v, didx_v, brows_v, drows_v, bsem, dsem):
    wid = lax.axis_index("s") * _NC + lax.axis_index("c")
    base = wid * _BPW

    pltpu.sync_copy(blk_idx_hbm.at[pl.ds(wid * _NCHUNK, _NCHUNK)], bidx_v)
    pltpu.sync_copy(dir_idx_hbm.at[pl.ds(wid * _NCHUNK, _NCHUNK)], didx_v)

    copies = []
    for j in range(_NCHUNK):
        copies.append(pltpu.async_copy(
            blk_tab_hbm.at[bidx_v.at[j]],
            brows_v.at[pl.ds(j * _CHUNK, _CHUNK)], bsem))
        copies.append(pltpu.async_copy(
            dir_tab_hbm.at[didx_v.at[j]],
            drows_v.at[pl.ds(j * _CHUNK, _CHUNK)], dsem))
    for c in copies:
        c.wait()

    pltpu.sync_copy(brows_v, out_hbm.at[pl.ds(base, _BPW), 0])
    pltpu.sync_copy(drows_v, out_hbm.at[pl.ds(base, _BPW), 1])


def kernel(block, direction, block_table, direction_table):
    blk = block.reshape(_NW * _NCHUNK, _CHUNK).astype(jnp.int32)
    dire = direction.reshape(_NW * _NCHUNK, _CHUNK).astype(jnp.int32)
    out = _action_encoder(blk, dire, block_table, direction_table)
    return out.reshape(BATCH, 2 * EMB)


# speedup vs baseline: 1.0502x; 1.0502x over previous
import functools

import jax
import jax.numpy as jnp
from jax import lax
from jax.experimental import pallas as pl
from jax.experimental.pallas import tpu as pltpu
from jax.experimental.pallas import tpu_sc as plsc

BATCH = 16384
EMB = 32

_info = plsc.get_sparse_core_info()
_NC = _info.num_cores
_NS = _info.num_subcores
_NW = _NC * _NS
_BPW = BATCH // _NW
_L = 16
_CHUNK = 128
_NCHUNK = _BPW // _CHUNK


@functools.partial(
    pl.kernel,
    mesh=plsc.VectorSubcoreMesh(core_axis_name="c", subcore_axis_name="s"),
    out_type=jax.ShapeDtypeStruct((2 * EMB, BATCH), jnp.float32),
    compiler_params=pltpu.CompilerParams(use_tc_tiling_on_sc=False,
                                         needs_layout_passes=False),
    scratch_types=[
        pltpu.VMEM((_NCHUNK, _CHUNK), jnp.int32),
        pltpu.VMEM((_NCHUNK, _CHUNK), jnp.int32),
        pltpu.VMEM((_BPW, EMB), jnp.float32),
        pltpu.VMEM((_BPW, EMB), jnp.float32),
        pltpu.VMEM((2 * EMB, _BPW), jnp.float32),
        pltpu.SemaphoreType.DMA,
        pltpu.SemaphoreType.DMA,
    ],
)
def _action_encoder(blk_idx_hbm, dir_idx_hbm, blk_tab_hbm, dir_tab_hbm,
                    out_hbm, bidx_v, didx_v, brows_v, drows_v, tbuf_v,
                    bsem, dsem):
    wid = lax.axis_index("s") * _NC + lax.axis_index("c")
    base = wid * _BPW

    pltpu.sync_copy(blk_idx_hbm.at[pl.ds(wid * _NCHUNK, _NCHUNK)], bidx_v)
    pltpu.sync_copy(dir_idx_hbm.at[pl.ds(wid * _NCHUNK, _NCHUNK)], didx_v)

    copies = []
    for j in range(_NCHUNK):
        copies.append(pltpu.async_copy(
            blk_tab_hbm.at[bidx_v.at[j]],
            brows_v.at[pl.ds(j * _CHUNK, _CHUNK)], bsem))
        copies.append(pltpu.async_copy(
            dir_tab_hbm.at[didx_v.at[j]],
            drows_v.at[pl.ds(j * _CHUNK, _CHUNK)], dsem))
    for c in copies:
        c.wait()

    def xpose(c, _):
        rv = jnp.arange(_L, dtype=jnp.int32) + c * _L
        for j in range(EMB):
            jv = jnp.full((_L,), j, dtype=jnp.int32)
            tbuf_v[j, pl.ds(c * _L, _L)] = plsc.load_gather(
                brows_v, [rv, jv])
            tbuf_v[EMB + j, pl.ds(c * _L, _L)] = plsc.load_gather(
                drows_v, [rv, jv])
        return _
    lax.fori_loop(0, _BPW // _L, xpose, None)

    pltpu.sync_copy(tbuf_v, out_hbm.at[:, pl.ds(base, _BPW)])


def kernel(block, direction, block_table, direction_table):
    blk = block.reshape(_NW * _NCHUNK, _CHUNK).astype(jnp.int32)
    dire = direction.reshape(_NW * _NCHUNK, _CHUNK).astype(jnp.int32)
    out_t = _action_encoder(blk, dire, block_table, direction_table)
    return out_t.T


# speedup vs baseline: 3.1130x; 2.9641x over previous
import functools

import jax
import jax.numpy as jnp
from jax import lax
from jax.experimental import pallas as pl
from jax.experimental.pallas import tpu as pltpu
from jax.experimental.pallas import tpu_sc as plsc

BATCH = 16384
EMB = 32
NBLK = 1000001
NDIR = 1002
_LANES = 128
_TAIL0 = (NBLK // _LANES) * _LANES
_NTAIL = NBLK - _TAIL0

_info = plsc.get_sparse_core_info()
_NC = _info.num_cores
_NS = _info.num_subcores
_NW = _NC * _NS
_BPW = BATCH // _NW
_L = 16
_CHUNK = 128
_NCHUNK = _BPW // _CHUNK
_NSLOT = 8


@functools.partial(
    pl.kernel,
    mesh=plsc.VectorSubcoreMesh(core_axis_name="c", subcore_axis_name="s"),
    out_type=jax.ShapeDtypeStruct((2 * EMB, BATCH), jnp.float32),
    compiler_params=pltpu.CompilerParams(needs_layout_passes=False),
    scratch_types=[
        pltpu.VMEM((_BPW,), jnp.int32),
        pltpu.VMEM((_BPW,), jnp.int32),
        pltpu.VMEM((_NSLOT, EMB, _LANES), jnp.float32),
        pltpu.VMEM((EMB, NDIR), jnp.float32),
        pltpu.VMEM((_NTAIL, EMB), jnp.float32),
        pltpu.VMEM((2 * EMB, _BPW), jnp.float32),
        pltpu.SemaphoreType.DMA,
        pltpu.SemaphoreType.DMA,
    ],
)
def _action_encoder(blk_idx_hbm, dir_idx_hbm, blk_t_hbm, dir_t_hbm,
                    tail_hbm, out_hbm, bidx_v, didx_v,
                    strips_v, dtab_v, tail_v, obuf_v, gsem, dsem):
    wid = lax.axis_index("s") * _NC + lax.axis_index("c")
    base = wid * _BPW

    pltpu.sync_copy(blk_idx_hbm.at[pl.ds(base, _BPW)], bidx_v)
    pltpu.sync_copy(dir_idx_hbm.at[pl.ds(base, _BPW)], didx_v)
    dcopies = [pltpu.async_copy(tail_hbm, tail_v, dsem),
               pltpu.async_copy(dir_t_hbm, dtab_v, dsem)]

    f0 = lax.iota(jnp.int32, _L)
    zeros = jnp.zeros((_L,), jnp.int32)

    def batch(t, _):
        k0 = t * _L
        iv = jnp.minimum(bidx_v[pl.ds(k0, _L)], _TAIL0 - 1)
        for half in range(2):
            copies = []
            for s in range(_NSLOT):
                i = iv[half * _NSLOT + s]
                col = pl.multiple_of((i >> 7) * _LANES, _LANES)
                copies.append(pltpu.async_copy(
                    blk_t_hbm.at[:, pl.ds(col, _LANES)], strips_v.at[s],
                    gsem))
            for c in copies:
                c.wait()
            for s in range(_NSLOT):
                j = half * _NSLOT + s
                i = iv[j]
                lv = zeros + (i & (_LANES - 1))
                kv = zeros + (k0 + j)
                sv = zeros + s
                g0 = plsc.load_gather(strips_v, [sv, f0, lv])
                g1 = plsc.load_gather(strips_v, [sv, f0 + _L, lv])
                plsc.store_scatter(obuf_v, [f0, kv], g0)
                plsc.store_scatter(obuf_v, [f0 + _L, kv], g1)
        return _
    lax.fori_loop(0, _BPW // _L, batch, None)

    for c in dcopies:
        c.wait()

    def dir_chunk(c, _):
        di = didx_v[pl.ds(c * _L, _L)]
        for f in range(EMB):
            fv = jnp.zeros((_L,), jnp.int32) + f
            obuf_v[EMB + f, pl.ds(c * _L, _L)] = plsc.load_gather(
                dtab_v, [fv, di])
        return _
    lax.fori_loop(0, _BPW // _L, dir_chunk, None)

    def fixup(c, _):
        kv = lax.iota(jnp.int32, _L) + c * _L
        bv = bidx_v[pl.ds(c * _L, _L)]
        wv = bv - _TAIL0
        m = wv >= 0
        wc = jnp.maximum(wv, 0)
        for f in range(EMB):
            fv = jnp.zeros((_L,), jnp.int32) + f
            vals = plsc.load_gather(tail_v, [wc, fv])
            plsc.store_scatter(obuf_v, [fv, kv], vals, mask=m)
        return _
    lax.fori_loop(0, _BPW // _L, fixup, None)

    pltpu.sync_copy(obuf_v, out_hbm.at[:, pl.ds(base, _BPW)])


def kernel(block, direction, block_table, direction_table):
    blk = block.reshape(BATCH).astype(jnp.int32)
    dire = direction.reshape(BATCH).astype(jnp.int32)
    tail = block_table[_TAIL0:]
    out_t = _action_encoder(blk, dire, block_table.T, direction_table.T,
                            tail)
    return out_t.T
